# initial kernel scaffold (unmeasured)
import jax
import jax.numpy as jnp
from jax import lax
from jax.experimental import pallas as pl
from jax.experimental.pallas import tpu as pltpu


def kernel(
    x,
):
    def body(*refs):
        pass

    out_shape = jax.ShapeDtypeStruct(..., jnp.float32)
    return pl.pallas_call(body, out_shape=out_shape)(...)


# baseline (device time: 32020 ns/iter reference)
import jax
import jax.numpy as jnp
from jax import lax
from jax.experimental import pallas as pl
from jax.experimental.pallas import tpu as pltpu


def kernel(x):
    m, n = x.shape
    half_n = n // 2

    def body(x_ref, out_ref, send_buf, send_sem, recv_sem):
        my_x = lax.axis_index("x")
        my_y = lax.axis_index("y")
        my_z = lax.axis_index("z")
        partner = (1 - my_x, my_y, my_z)

        barrier_sem = pltpu.get_barrier_semaphore()
        pl.semaphore_signal(
            barrier_sem, inc=1,
            device_id=partner, device_id_type=pl.DeviceIdType.MESH,
        )
        pl.semaphore_wait(barrier_sem, 1)

        send_buf[:, :] = x_ref[:, pl.ds((1 - my_x) * half_n, half_n)].astype(
            jnp.bfloat16
        )
        rdma = pltpu.make_async_remote_copy(
            src_ref=send_buf,
            dst_ref=out_ref.at[pl.ds(my_x * m, m), :],
            send_sem=send_sem,
            recv_sem=recv_sem,
            device_id=partner,
            device_id_type=pl.DeviceIdType.MESH,
        )
        rdma.start()

        out_ref[pl.ds(my_x * m, m), :] = x_ref[
            :, pl.ds(my_x * half_n, half_n)
        ].astype(jnp.bfloat16)

        rdma.wait()

    return pl.pallas_call(
        body,
        out_shape=jax.ShapeDtypeStruct((2 * m, half_n), jnp.bfloat16),
        in_specs=[pl.BlockSpec(memory_space=pltpu.VMEM)],
        out_specs=pl.BlockSpec(memory_space=pltpu.VMEM),
        scratch_shapes=[
            pltpu.VMEM((m, half_n), jnp.bfloat16),
            pltpu.SemaphoreType.DMA,
            pltpu.SemaphoreType.DMA,
        ],
        compiler_params=pltpu.CompilerParams(collective_id=0),
    )(x)


# device time: 22823 ns/iter; 1.4030x vs baseline; 1.4030x over previous
import jax
import jax.numpy as jnp
from jax import lax
from jax.experimental import pallas as pl
from jax.experimental.pallas import tpu as pltpu

C = 4
S2C = 2


def kernel(x):
    m, n = x.shape
    hn = n // 2
    qrows = m // 4
    crows = qrows // C

    def body(
        x_ref, out_ref, stage,
        imp_s, imp_r, s1y_s, s1y_r, s1z_s, s1z_r,
        s2y_s, s2y_r, s2z_s, s2z_r,
    ):
        my_x = lax.axis_index("x")
        my_y = lax.axis_index("y")
        my_z = lax.axis_index("z")
        px = (1 - my_x, my_y, my_z)
        py = (my_x, 1 - my_y, my_z)
        pz = (my_x, my_y, 1 - my_z)
        qid = 2 * my_y + my_z
        qid_y = 2 * (1 - my_y) + my_z
        qid_z = 2 * my_y + (1 - my_z)

        barrier_sem = pltpu.get_barrier_semaphore()
        for nbr in (px, py, pz):
            pl.semaphore_signal(
                barrier_sem, inc=1,
                device_id=nbr, device_id_type=pl.DeviceIdType.MESH,
            )
        pl.semaphore_wait(barrier_sem, 3)

        rem_base = (1 - my_x) * m
        snd_base = my_x * m

        def copy(src_rows, dst_rows, nrows, ssem, rsem, nbr, src=None):
            return pltpu.make_async_remote_copy(
                src_ref=(out_ref if src is None else src).at[
                    pl.ds(src_rows, nrows), :
                ],
                dst_ref=out_ref.at[pl.ds(dst_rows, nrows), :],
                send_sem=ssem,
                recv_sem=rsem,
                device_id=nbr,
                device_id_type=pl.DeviceIdType.MESH,
            )

        imp = []
        for i in range(C):
            stage[i, :, :] = x_ref[
                pl.ds(qid * qrows + i * crows, crows),
                pl.ds((1 - my_x) * hn, hn),
            ].astype(jnp.bfloat16)
            rd = pltpu.make_async_remote_copy(
                src_ref=stage.at[i],
                dst_ref=out_ref.at[
                    pl.ds(snd_base + qid * qrows + i * crows, crows), :
                ],
                send_sem=imp_s.at[i],
                recv_sem=imp_r.at[i],
                device_id=px,
                device_id_type=pl.DeviceIdType.MESH,
            )
            rd.start()
            imp.append(rd)

        out_ref[pl.ds(my_x * m, m), :] = x_ref[
            :, pl.ds(my_x * hn, hn)
        ].astype(jnp.bfloat16)

        s1y, s1z = [], []
        for i in range(C):
            imp[i].wait_recv()
            r = rem_base + qid * qrows + i * crows
            rdy = copy(r, r, crows, s1y_s.at[i], s1y_r.at[i], py)
            rdy.start()
            s1y.append(rdy)
            rdz = copy(r, r, crows, s1z_s.at[i], s1z_r.at[i], pz)
            rdz.start()
            s1z.append(rdz)

        s2y, s2z = [], []
        for i in range(C):
            s1z[i].wait_recv()
            if i < S2C:
                r = rem_base + qid_z * qrows + i * crows
                rd = copy(r, r, crows, s2y_s.at[i], s2y_r.at[i], py)
                rd.start()
                s2y.append(rd)
            s1y[i].wait_recv()
            if i >= C - S2C:
                j = i - (C - S2C)
                r = rem_base + qid_y * qrows + i * crows
                rd = copy(r, r, crows, s2z_s.at[j], s2z_r.at[j], pz)
                rd.start()
                s2z.append(rd)

        for j in range(S2C):
            s2y[j].wait_recv()
            s2z[j].wait_recv()

        for rd in imp + s1y + s1z + s2y + s2z:
            rd.wait_send()

    return pl.pallas_call(
        body,
        out_shape=jax.ShapeDtypeStruct((2 * m, hn), jnp.bfloat16),
        in_specs=[pl.BlockSpec(memory_space=pltpu.VMEM)],
        out_specs=pl.BlockSpec(memory_space=pltpu.VMEM),
        scratch_shapes=[
            pltpu.VMEM((C, crows, hn), jnp.bfloat16),
            pltpu.SemaphoreType.DMA((C,)),
            pltpu.SemaphoreType.DMA((C,)),
            pltpu.SemaphoreType.DMA((C,)),
            pltpu.SemaphoreType.DMA((C,)),
            pltpu.SemaphoreType.DMA((C,)),
            pltpu.SemaphoreType.DMA((C,)),
            pltpu.SemaphoreType.DMA((S2C,)),
            pltpu.SemaphoreType.DMA((S2C,)),
            pltpu.SemaphoreType.DMA((S2C,)),
            pltpu.SemaphoreType.DMA((S2C,)),
        ],
        compiler_params=pltpu.CompilerParams(collective_id=0),
    )(x)
